# Initial kernel scaffold; baseline (speedup 1.0000x reference)
import jax
import jax.numpy as jnp
from jax.experimental import pallas as pl


def kernel(tok, emb, W1, b1, W2, b2):
    raise NotImplementedError("write your pallas kernel here")


# speedup vs baseline: 3.5174x; 3.5174x over previous
import functools

import jax
import jax.numpy as jnp
from jax import lax
from jax.experimental import pallas as pl
from jax.experimental.pallas import tpu as pltpu
from jax.experimental.pallas import tpu_sc as plsc

VOCAB = 100000
D = 128
NQ = 64
ROW_BLK = 2000

NW = 32
CHUNK = 128
FIRE = 8


def _vocab_mlp_kernel(emb_ref, w1_ref, b1_ref, w2_ref, b2_ref, p_ref):
    x = emb_ref[...]
    h = jnp.tanh(
        jnp.dot(x, w1_ref[...], preferred_element_type=jnp.float32,
                precision=lax.Precision.HIGHEST)
        + b1_ref[...]
    )
    p_ref[...] = (
        jnp.dot(h, w2_ref[...], preferred_element_type=jnp.float32,
                precision=lax.Precision.HIGHEST)
        + b2_ref[...]
    )


def _vocab_mlp(emb, W1, b1, W2, b2):
    grid = VOCAB // ROW_BLK
    return pl.pallas_call(
        _vocab_mlp_kernel,
        grid=(grid,),
        in_specs=[
            pl.BlockSpec((ROW_BLK, D), lambda i: (i, 0)),
            pl.BlockSpec((D, D), lambda i: (0, 0)),
            pl.BlockSpec((1, D), lambda i: (0, 0)),
            pl.BlockSpec((D, NQ), lambda i: (0, 0)),
            pl.BlockSpec((1, NQ), lambda i: (0, 0)),
        ],
        out_specs=pl.BlockSpec((ROW_BLK, NQ), lambda i: (i, 0)),
        out_shape=jax.ShapeDtypeStruct((VOCAB, NQ), jnp.float32),
    )(emb, W1, b1.reshape(1, D), W2, b2.reshape(1, NQ))


def _make_sc_gather(n_tokens):
    per_w = n_tokens // NW
    n_chunks = per_w // CHUNK
    n_steps = n_chunks // FIRE
    idx_rows = per_w // CHUNK

    mesh = plsc.VectorSubcoreMesh(core_axis_name="c", subcore_axis_name="s")
    info = plsc.get_sparse_core_info()
    nc = info.num_cores

    @functools.partial(
        pl.kernel,
        out_type=jax.ShapeDtypeStruct((n_tokens, NQ), jnp.float32),
        mesh=mesh,
        scratch_types=[
            pltpu.VMEM((idx_rows, CHUNK), jnp.int32),
            pltpu.VMEM((FIRE * CHUNK, NQ), jnp.float32),
            pltpu.SemaphoreType.DMA,
        ],
        compiler_params=pltpu.CompilerParams(use_tc_tiling_on_sc=False),
    )
    def gather_kernel(table_hbm, idx_hbm, out_hbm, idx_v, rows_v, sem):
        wid = lax.axis_index("s") * nc + lax.axis_index("c")
        base = wid * per_w
        pltpu.sync_copy(idx_hbm.at[pl.ds(wid * idx_rows, idx_rows)], idx_v)

        def step(g, carry):
            copies = []
            for b in range(FIRE):
                j = g * FIRE + b
                copies.append(
                    pltpu.async_copy(
                        table_hbm.at[idx_v.at[j]],
                        rows_v.at[pl.ds(b * CHUNK, CHUNK)],
                        sem,
                    )
                )
            for c in copies:
                c.wait()
            pltpu.sync_copy(
                rows_v,
                out_hbm.at[pl.ds(base + g * (FIRE * CHUNK), FIRE * CHUNK)],
            )
            return carry

        lax.fori_loop(0, n_steps, step, 0)

    return gather_kernel


def kernel(tok, emb, W1, b1, W2, b2):
    B, L = tok.shape
    n_tokens = B * L
    table = _vocab_mlp(emb, W1, b1, W2, b2)
    idx2d = tok.reshape(n_tokens // CHUNK, CHUNK).astype(jnp.int32)
    out = _make_sc_gather(n_tokens)(table, idx2d)
    return out.reshape(B, L, NQ)
